# baseline (speedup 1.0000x reference)
import functools

import jax
import jax.numpy as jnp
from jax.experimental import pallas as pl

_K = 9
_G = 8
_QB = 512
_N = 4096
_B = 4
_C = 9


def _knn_feat_kernel(center_ref, qblk_ref, out_ref):
    keys = center_ref[0]
    sqk = jnp.sum(keys * keys, axis=0, keepdims=True)

    q = qblk_ref[0]
    sqq = jnp.sum(q * q, axis=0, keepdims=True)
    sqq_t = jnp.transpose(sqq)

    dqk = jax.lax.dot_general(
        q.astype(jnp.bfloat16), keys.astype(jnp.bfloat16),
        (((0,), (0,)), ((), ())),
        preferred_element_type=jnp.float32)
    dist = (sqq_t + sqk) - 2.0 * dqk

    iota = jax.lax.broadcasted_iota(jnp.int32, (_QB, _N), 1)
    inf = jnp.float32(jnp.inf)

    rel = []
    for t in range(_K):
        j = jnp.argmin(dist, axis=1)[:, None]
        sel = iota == j
        if t > 0:
            oh = sel.astype(jnp.float32)
            coords = jax.lax.dot_general(
                keys, oh, (((1,), (1,)), ((), ())),
                preferred_element_type=jnp.float32,
                precision=jax.lax.Precision.HIGHEST)
            rel.append(coords - q)
        if t < _K - 1:
            dist = jnp.where(sel, inf, dist)

    nx = jnp.concatenate([r[0:1] for r in rel], axis=0)
    ny = jnp.concatenate([r[1:2] for r in rel], axis=0)
    nz = jnp.concatenate([r[2:3] for r in rel], axis=0)

    den = jnp.abs(nx) + jnp.abs(ny)
    a = nx / jnp.where(den == 0.0, 1.0, den)
    p = jnp.where(ny >= 0.0, 1.0 - a, a - 1.0)
    p = jnp.where(den == 0.0, 0.0, p)

    ranks = []
    for i in range(_G):
        pi = p[i:i + 1]
        r_i = jnp.sum((p < pi).astype(jnp.int32), axis=0, keepdims=True)
        if i > 0:
            r_i = r_i + jnp.sum((p[0:i] == pi).astype(jnp.int32),
                                axis=0, keepdims=True)
        ranks.append(r_i)
    rank = jnp.concatenate(ranks, axis=0)

    def permute(v):
        rows = []
        for r in range(_G):
            selr = rank == r
            rows.append(jnp.sum(jnp.where(selr, v, 0.0), axis=0,
                                keepdims=True))
        return jnp.concatenate(rows, axis=0)

    sx, sy, sz = permute(nx), permute(ny), permute(nz)
    rx = jnp.concatenate([sx[1:], sx[0:1]], axis=0)
    ry = jnp.concatenate([sy[1:], sy[0:1]], axis=0)
    rz = jnp.concatenate([sz[1:], sz[0:1]], axis=0)

    cxn = sy * rz - sz * ry
    cyn = sz * rx - sx * rz
    czn = sx * ry - sy * rx
    nrm = jnp.sqrt(cxn * cxn + cyn * cyn + czn * czn)
    ux = cxn / nrm
    uy = cyn / nrm
    uz = czn / nrm
    posm = jnp.where(ux[0:1] > 0.0, 1.0, -1.0)
    gx, gy, gz = ux * posm, uy * posm, uz * posm

    ccx = (sx + rx) / 3.0
    ccy = (sy + ry) / 3.0
    ccz = (sz + rz) / 3.0

    rho = jnp.sqrt(ccx * ccx + ccy * ccy + ccz * ccz)
    rho_safe = jnp.where(rho == 0.0, 1.0, rho)
    ratio = jnp.clip(ccz / rho_safe, -1.0, 1.0)
    acos = jnp.arctan2(jnp.sqrt(jnp.maximum((1.0 - ratio) * (1.0 + ratio),
                                            0.0)), ratio)
    theta = jnp.where(rho == 0.0, 0.0, acos) * (1.0 / jnp.pi)
    phi = jnp.arctan2(ccy, ccx) * (1.0 / (2.0 * jnp.pi)) + 0.5

    nanm = (gx != gx) | (gy != gy) | (gz != gz)
    slot = jax.lax.broadcasted_iota(jnp.int32, (_G, _QB), 0)
    first = jnp.min(jnp.where(nanm, _G, slot), axis=0, keepdims=True)
    first = jnp.where(first == _G, 0, first)
    fsel = slot == first

    def first_val(v):
        return jnp.sum(jnp.where(fsel, v, 0.0), axis=0, keepdims=True)

    gx = jnp.where(nanm, first_val(gx), gx)
    gy = jnp.where(nanm, first_val(gy), gy)
    gz = jnp.where(nanm, first_val(gz), gz)
    ccx = jnp.where(nanm, first_val(ccx), ccx)
    ccy = jnp.where(nanm, first_val(ccy), ccy)
    ccz = jnp.where(nanm, first_val(ccz), ccz)

    chans = (ccx, ccy, ccz, rho, theta, phi, gx, gy, gz)
    for c, arr in enumerate(chans):
        out_ref[c, 0] = arr


def _mlp_kernel(feat_ref, w1_ref, g1_ref, b1_ref, w2_ref, cb2_ref, g2_ref,
                b2_ref, w3_ref, cb3_ref, out_ref):
    f = feat_ref[...]

    def conv(w, x):
        return jax.lax.dot_general(
            w.astype(jnp.bfloat16), x.astype(jnp.bfloat16),
            (((1,), (0,)), ((), ())),
            preferred_element_type=jnp.float32)

    def bn(x, g, b):
        m = jnp.mean(x, axis=1, keepdims=True)
        v = jnp.mean((x - m) * (x - m), axis=1, keepdims=True)
        xn = (x - m) / jnp.sqrt(v + 1e-5)
        return xn * g + b

    x = jax.nn.relu(bn(conv(w1_ref[...], f), g1_ref[...], b1_ref[...]))
    x = jax.nn.relu(bn(conv(w2_ref[...], x) + cb2_ref[...],
                       g2_ref[...], b2_ref[...]))
    x = conv(w3_ref[...], x) + cb3_ref[...]

    for b in range(_B):
        acc = x[:, b * _G * _N:(b * _G) * _N + _N]
        for g in range(1, _G):
            base = (b * _G + g) * _N
            acc = acc + x[:, base:base + _N]
        out_ref[b] = acc


@jax.jit
def kernel(center, conv1_w, bn1_g, bn1_b, conv2_w, conv2_b, bn2_g, bn2_b,
           conv3_w, conv3_b):
    feat = pl.pallas_call(
        _knn_feat_kernel,
        grid=(_B, _N // _QB),
        in_specs=[pl.BlockSpec((1, 3, _N), lambda b, q: (b, 0, 0)),
                  pl.BlockSpec((1, 3, _QB), lambda b, q: (b, 0, q))],
        out_specs=pl.BlockSpec((_C, 1, _G, _QB), lambda b, q: (0, b, 0, q)),
        out_shape=jax.ShapeDtypeStruct((_C, _B, _G, _N), jnp.float32),
    )(center, center)

    feat2 = feat.reshape(_C, _B * _G * _N)
    full = lambda s: pl.BlockSpec(s, lambda: tuple(0 for _ in s))
    out = pl.pallas_call(
        _mlp_kernel,
        grid=(),
        in_specs=[full((_C, _B * _G * _N))] + [full((_C, _C)), full((_C, 1)),
                  full((_C, 1)), full((_C, _C)), full((_C, 1)), full((_C, 1)),
                  full((_C, 1)), full((_C, _C)), full((_C, 1))],
        out_specs=full((_B, _C, _N)),
        out_shape=jax.ShapeDtypeStruct((_B, _C, _N), jnp.float32),
    )(feat2, conv1_w, bn1_g.reshape(_C, 1), bn1_b.reshape(_C, 1),
      conv2_w, conv2_b.reshape(_C, 1), bn2_g.reshape(_C, 1),
      bn2_b.reshape(_C, 1), conv3_w, conv3_b.reshape(_C, 1))
    return out


# speedup vs baseline: 1.1449x; 1.1449x over previous
import functools

import jax
import jax.numpy as jnp
from jax.experimental import pallas as pl

_K = 9
_G = 8
_QB = 128
_N = 4096
_B = 4
_C = 9


def _knn_feat_kernel(center_ref, qblk_ref, out_ref):
    keys = center_ref[0]
    sqk = jnp.sum(keys * keys, axis=0, keepdims=True)

    q = qblk_ref[0]
    sqq = jnp.sum(q * q, axis=0, keepdims=True)
    sqq_t = jnp.transpose(sqq)

    dqk = jax.lax.dot_general(
        q.astype(jnp.bfloat16), keys.astype(jnp.bfloat16),
        (((0,), (0,)), ((), ())),
        preferred_element_type=jnp.float32)
    dist = (sqq_t + sqk) - 2.0 * dqk

    iota = jax.lax.broadcasted_iota(jnp.int32, (_QB, _N), 1)
    inf = jnp.float32(jnp.inf)

    rel = []
    for t in range(_K):
        j = jnp.argmin(dist, axis=1)[:, None]
        sel = iota == j
        if t > 0:
            oh = sel.astype(jnp.float32)
            coords = jax.lax.dot_general(
                keys, oh, (((1,), (1,)), ((), ())),
                preferred_element_type=jnp.float32,
                precision=jax.lax.Precision.HIGHEST)
            rel.append(coords - q)
        if t < _K - 1:
            dist = jnp.where(sel, inf, dist)

    nx = jnp.concatenate([r[0:1] for r in rel], axis=0)
    ny = jnp.concatenate([r[1:2] for r in rel], axis=0)
    nz = jnp.concatenate([r[2:3] for r in rel], axis=0)

    den = jnp.abs(nx) + jnp.abs(ny)
    a = nx / jnp.where(den == 0.0, 1.0, den)
    p = jnp.where(ny >= 0.0, 1.0 - a, a - 1.0)
    p = jnp.where(den == 0.0, 0.0, p)

    ranks = []
    for i in range(_G):
        pi = p[i:i + 1]
        r_i = jnp.sum((p < pi).astype(jnp.int32), axis=0, keepdims=True)
        if i > 0:
            r_i = r_i + jnp.sum((p[0:i] == pi).astype(jnp.int32),
                                axis=0, keepdims=True)
        ranks.append(r_i)
    rank = jnp.concatenate(ranks, axis=0)

    def permute(v):
        rows = []
        for r in range(_G):
            selr = rank == r
            rows.append(jnp.sum(jnp.where(selr, v, 0.0), axis=0,
                                keepdims=True))
        return jnp.concatenate(rows, axis=0)

    sx, sy, sz = permute(nx), permute(ny), permute(nz)
    rx = jnp.concatenate([sx[1:], sx[0:1]], axis=0)
    ry = jnp.concatenate([sy[1:], sy[0:1]], axis=0)
    rz = jnp.concatenate([sz[1:], sz[0:1]], axis=0)

    cxn = sy * rz - sz * ry
    cyn = sz * rx - sx * rz
    czn = sx * ry - sy * rx
    nrm = jnp.sqrt(cxn * cxn + cyn * cyn + czn * czn)
    ux = cxn / nrm
    uy = cyn / nrm
    uz = czn / nrm
    posm = jnp.where(ux[0:1] > 0.0, 1.0, -1.0)
    gx, gy, gz = ux * posm, uy * posm, uz * posm

    ccx = (sx + rx) / 3.0
    ccy = (sy + ry) / 3.0
    ccz = (sz + rz) / 3.0

    rho = jnp.sqrt(ccx * ccx + ccy * ccy + ccz * ccz)
    rho_safe = jnp.where(rho == 0.0, 1.0, rho)
    ratio = jnp.clip(ccz / rho_safe, -1.0, 1.0)
    acos = jnp.arctan2(jnp.sqrt(jnp.maximum((1.0 - ratio) * (1.0 + ratio),
                                            0.0)), ratio)
    theta = jnp.where(rho == 0.0, 0.0, acos) * (1.0 / jnp.pi)
    phi = jnp.arctan2(ccy, ccx) * (1.0 / (2.0 * jnp.pi)) + 0.5

    nanm = (gx != gx) | (gy != gy) | (gz != gz)
    slot = jax.lax.broadcasted_iota(jnp.int32, (_G, _QB), 0)
    first = jnp.min(jnp.where(nanm, _G, slot), axis=0, keepdims=True)
    first = jnp.where(first == _G, 0, first)
    fsel = slot == first

    def first_val(v):
        return jnp.sum(jnp.where(fsel, v, 0.0), axis=0, keepdims=True)

    gx = jnp.where(nanm, first_val(gx), gx)
    gy = jnp.where(nanm, first_val(gy), gy)
    gz = jnp.where(nanm, first_val(gz), gz)
    ccx = jnp.where(nanm, first_val(ccx), ccx)
    ccy = jnp.where(nanm, first_val(ccy), ccy)
    ccz = jnp.where(nanm, first_val(ccz), ccz)

    chans = (ccx, ccy, ccz, rho, theta, phi, gx, gy, gz)
    for c, arr in enumerate(chans):
        out_ref[c, 0] = arr


def _mlp_kernel(feat_ref, w1_ref, g1_ref, b1_ref, w2_ref, cb2_ref, g2_ref,
                b2_ref, w3_ref, cb3_ref, out_ref):
    f = feat_ref[...]

    def conv(w, x):
        return jax.lax.dot_general(
            w.astype(jnp.bfloat16), x.astype(jnp.bfloat16),
            (((1,), (0,)), ((), ())),
            preferred_element_type=jnp.float32)

    def bn(x, g, b):
        m = jnp.mean(x, axis=1, keepdims=True)
        v = jnp.mean((x - m) * (x - m), axis=1, keepdims=True)
        xn = (x - m) / jnp.sqrt(v + 1e-5)
        return xn * g + b

    x = jax.nn.relu(bn(conv(w1_ref[...], f), g1_ref[...], b1_ref[...]))
    x = jax.nn.relu(bn(conv(w2_ref[...], x) + cb2_ref[...],
                       g2_ref[...], b2_ref[...]))
    x = conv(w3_ref[...], x) + cb3_ref[...]

    for b in range(_B):
        acc = x[:, b * _G * _N:(b * _G) * _N + _N]
        for g in range(1, _G):
            base = (b * _G + g) * _N
            acc = acc + x[:, base:base + _N]
        out_ref[b] = acc


@jax.jit
def kernel(center, conv1_w, bn1_g, bn1_b, conv2_w, conv2_b, bn2_g, bn2_b,
           conv3_w, conv3_b):
    feat = pl.pallas_call(
        _knn_feat_kernel,
        grid=(_B, _N // _QB),
        in_specs=[pl.BlockSpec((1, 3, _N), lambda b, q: (b, 0, 0)),
                  pl.BlockSpec((1, 3, _QB), lambda b, q: (b, 0, q))],
        out_specs=pl.BlockSpec((_C, 1, _G, _QB), lambda b, q: (0, b, 0, q)),
        out_shape=jax.ShapeDtypeStruct((_C, _B, _G, _N), jnp.float32),
    )(center, center)

    feat2 = feat.reshape(_C, _B * _G * _N)
    full = lambda s: pl.BlockSpec(s, lambda: tuple(0 for _ in s))
    out = pl.pallas_call(
        _mlp_kernel,
        grid=(),
        in_specs=[full((_C, _B * _G * _N))] + [full((_C, _C)), full((_C, 1)),
                  full((_C, 1)), full((_C, _C)), full((_C, 1)), full((_C, 1)),
                  full((_C, 1)), full((_C, _C)), full((_C, 1))],
        out_specs=full((_B, _C, _N)),
        out_shape=jax.ShapeDtypeStruct((_B, _C, _N), jnp.float32),
    )(feat2, conv1_w, bn1_g.reshape(_C, 1), bn1_b.reshape(_C, 1),
      conv2_w, conv2_b.reshape(_C, 1), bn2_g.reshape(_C, 1),
      bn2_b.reshape(_C, 1), conv3_w, conv3_b.reshape(_C, 1))
    return out


# speedup vs baseline: 1.7070x; 1.4910x over previous
import functools

import jax
import jax.numpy as jnp
from jax.experimental import pallas as pl

_K = 9
_G = 8
_QB = 256
_N = 4096
_B = 4
_C = 9


def _knn_feat_kernel(center_ref, qblk_ref, out_ref):
    keys = center_ref[0]
    sqk = jnp.sum(keys * keys, axis=0, keepdims=True)

    q = qblk_ref[0]
    sqq = jnp.sum(q * q, axis=0, keepdims=True)
    sqq_t = jnp.transpose(sqq)

    dqk = jax.lax.dot_general(
        q.astype(jnp.bfloat16), keys.astype(jnp.bfloat16),
        (((0,), (0,)), ((), ())),
        preferred_element_type=jnp.float32)
    dist = (sqq_t + sqk) - 2.0 * dqk

    iota = jax.lax.broadcasted_iota(jnp.int32, (_QB, _N), 1)
    big = jnp.float32(3e38)

    k_hi = keys.astype(jnp.bfloat16)
    r1 = keys - k_hi.astype(jnp.float32)
    k_mid = r1.astype(jnp.bfloat16)
    k_lo = (r1 - k_mid.astype(jnp.float32)).astype(jnp.bfloat16)

    def extract(ohb):
        dn = (((1,), (1,)), ((), ()))
        c = jax.lax.dot_general(k_hi, ohb, dn,
                                preferred_element_type=jnp.float32)
        c = c + jax.lax.dot_general(k_mid, ohb, dn,
                                    preferred_element_type=jnp.float32)
        return c + jax.lax.dot_general(k_lo, ohb, dn,
                                       preferred_element_type=jnp.float32)

    rel = []
    for t in range(_K):
        j = jnp.argmin(dist, axis=1)[:, None]
        ohf = jnp.where(iota == j, 1.0, 0.0)
        if t > 0:
            rel.append(extract(ohf.astype(jnp.bfloat16)) - q)
        if t < _K - 1:
            dist = dist + ohf * big

    nx = jnp.concatenate([r[0:1] for r in rel], axis=0)
    ny = jnp.concatenate([r[1:2] for r in rel], axis=0)
    nz = jnp.concatenate([r[2:3] for r in rel], axis=0)

    den = jnp.abs(nx) + jnp.abs(ny)
    a = nx / jnp.where(den == 0.0, 1.0, den)
    p = jnp.where(ny >= 0.0, 1.0 - a, a - 1.0)
    p = jnp.where(den == 0.0, 0.0, p)

    ranks = []
    for i in range(_G):
        pi = p[i:i + 1]
        r_i = jnp.sum((p < pi).astype(jnp.int32), axis=0, keepdims=True)
        if i > 0:
            r_i = r_i + jnp.sum((p[0:i] == pi).astype(jnp.int32),
                                axis=0, keepdims=True)
        ranks.append(r_i)
    rank = jnp.concatenate(ranks, axis=0)

    def permute(v):
        rows = []
        for r in range(_G):
            selr = rank == r
            rows.append(jnp.sum(jnp.where(selr, v, 0.0), axis=0,
                                keepdims=True))
        return jnp.concatenate(rows, axis=0)

    sx, sy, sz = permute(nx), permute(ny), permute(nz)
    rx = jnp.concatenate([sx[1:], sx[0:1]], axis=0)
    ry = jnp.concatenate([sy[1:], sy[0:1]], axis=0)
    rz = jnp.concatenate([sz[1:], sz[0:1]], axis=0)

    cxn = sy * rz - sz * ry
    cyn = sz * rx - sx * rz
    czn = sx * ry - sy * rx
    nrm = jnp.sqrt(cxn * cxn + cyn * cyn + czn * czn)
    ux = cxn / nrm
    uy = cyn / nrm
    uz = czn / nrm
    posm = jnp.where(ux[0:1] > 0.0, 1.0, -1.0)
    gx, gy, gz = ux * posm, uy * posm, uz * posm

    ccx = (sx + rx) / 3.0
    ccy = (sy + ry) / 3.0
    ccz = (sz + rz) / 3.0

    rho = jnp.sqrt(ccx * ccx + ccy * ccy + ccz * ccz)
    rho_safe = jnp.where(rho == 0.0, 1.0, rho)
    ratio = jnp.clip(ccz / rho_safe, -1.0, 1.0)
    acos = jnp.arctan2(jnp.sqrt(jnp.maximum((1.0 - ratio) * (1.0 + ratio),
                                            0.0)), ratio)
    theta = jnp.where(rho == 0.0, 0.0, acos) * (1.0 / jnp.pi)
    phi = jnp.arctan2(ccy, ccx) * (1.0 / (2.0 * jnp.pi)) + 0.5

    nanm = (gx != gx) | (gy != gy) | (gz != gz)
    slot = jax.lax.broadcasted_iota(jnp.int32, (_G, _QB), 0)
    first = jnp.min(jnp.where(nanm, _G, slot), axis=0, keepdims=True)
    first = jnp.where(first == _G, 0, first)
    fsel = slot == first

    def first_val(v):
        return jnp.sum(jnp.where(fsel, v, 0.0), axis=0, keepdims=True)

    gx = jnp.where(nanm, first_val(gx), gx)
    gy = jnp.where(nanm, first_val(gy), gy)
    gz = jnp.where(nanm, first_val(gz), gz)
    ccx = jnp.where(nanm, first_val(ccx), ccx)
    ccy = jnp.where(nanm, first_val(ccy), ccy)
    ccz = jnp.where(nanm, first_val(ccz), ccz)

    chans = (ccx, ccy, ccz, rho, theta, phi, gx, gy, gz)
    for c, arr in enumerate(chans):
        out_ref[c, 0] = arr


def _mlp_kernel(feat_ref, w1_ref, g1_ref, b1_ref, w2_ref, cb2_ref, g2_ref,
                b2_ref, w3_ref, cb3_ref, out_ref):
    f = feat_ref[...]

    def conv(w, x):
        return jax.lax.dot_general(
            w.astype(jnp.bfloat16), x.astype(jnp.bfloat16),
            (((1,), (0,)), ((), ())),
            preferred_element_type=jnp.float32)

    def bn(x, g, b):
        m = jnp.mean(x, axis=1, keepdims=True)
        v = jnp.mean((x - m) * (x - m), axis=1, keepdims=True)
        xn = (x - m) / jnp.sqrt(v + 1e-5)
        return xn * g + b

    x = jax.nn.relu(bn(conv(w1_ref[...], f), g1_ref[...], b1_ref[...]))
    x = jax.nn.relu(bn(conv(w2_ref[...], x) + cb2_ref[...],
                       g2_ref[...], b2_ref[...]))
    x = conv(w3_ref[...], x) + cb3_ref[...]

    for b in range(_B):
        acc = x[:, b * _G * _N:(b * _G) * _N + _N]
        for g in range(1, _G):
            base = (b * _G + g) * _N
            acc = acc + x[:, base:base + _N]
        out_ref[b] = acc


@jax.jit
def kernel(center, conv1_w, bn1_g, bn1_b, conv2_w, conv2_b, bn2_g, bn2_b,
           conv3_w, conv3_b):
    feat = pl.pallas_call(
        _knn_feat_kernel,
        grid=(_B, _N // _QB),
        in_specs=[pl.BlockSpec((1, 3, _N), lambda b, q: (b, 0, 0)),
                  pl.BlockSpec((1, 3, _QB), lambda b, q: (b, 0, q))],
        out_specs=pl.BlockSpec((_C, 1, _G, _QB), lambda b, q: (0, b, 0, q)),
        out_shape=jax.ShapeDtypeStruct((_C, _B, _G, _N), jnp.float32),
    )(center, center)

    feat2 = feat.reshape(_C, _B * _G * _N)
    full = lambda s: pl.BlockSpec(s, lambda: tuple(0 for _ in s))
    out = pl.pallas_call(
        _mlp_kernel,
        grid=(),
        in_specs=[full((_C, _B * _G * _N))] + [full((_C, _C)), full((_C, 1)),
                  full((_C, 1)), full((_C, _C)), full((_C, 1)), full((_C, 1)),
                  full((_C, 1)), full((_C, _C)), full((_C, 1))],
        out_specs=full((_B, _C, _N)),
        out_shape=jax.ShapeDtypeStruct((_B, _C, _N), jnp.float32),
    )(feat2, conv1_w, bn1_g.reshape(_C, 1), bn1_b.reshape(_C, 1),
      conv2_w, conv2_b.reshape(_C, 1), bn2_g.reshape(_C, 1),
      bn2_b.reshape(_C, 1), conv3_w, conv3_b.reshape(_C, 1))
    return out


# speedup vs baseline: 2.1830x; 1.2789x over previous
import functools

import jax
import jax.numpy as jnp
from jax.experimental import pallas as pl

_K = 9
_G = 8
_QB = 256
_N = 4096
_B = 4
_C = 9


def _knn_feat_kernel(center_ref, qblk_ref, out_ref):
    keys = center_ref[0]
    sqk = jnp.sum(keys * keys, axis=0, keepdims=True)

    q = qblk_ref[0]
    sqq = jnp.sum(q * q, axis=0, keepdims=True)
    sqq_t = jnp.transpose(sqq)

    dqk = jax.lax.dot_general(
        q.astype(jnp.bfloat16), keys.astype(jnp.bfloat16),
        (((0,), (0,)), ((), ())),
        preferred_element_type=jnp.float32)
    dist = (sqq_t + sqk) - 2.0 * dqk

    iota = jax.lax.broadcasted_iota(jnp.int32, (_QB, _N), 1)
    big = jnp.float32(3e38)

    hi = keys.astype(jnp.bfloat16).astype(jnp.float32)
    r1 = keys - hi
    mid = r1.astype(jnp.bfloat16).astype(jnp.float32)
    r2 = r1 - mid
    k_t = jnp.transpose(
        jnp.concatenate([hi, mid, r2], axis=0)).astype(jnp.bfloat16)

    def extract(ohb):
        c = jax.lax.dot_general(ohb, k_t, (((1,), (0,)), ((), ())),
                                preferred_element_type=jnp.float32)
        return c[:, 0:3] + c[:, 3:6] + c[:, 6:9]

    rel = []
    for t in range(_K):
        j = jnp.argmin(dist, axis=1)[:, None]
        ohf = jnp.where(iota == j, 1.0, 0.0)
        if t > 0:
            rel.append(extract(ohf.astype(jnp.bfloat16)))
        if t < _K - 1:
            dist = dist + ohf * big

    qt = jnp.transpose(q)
    nx = jnp.concatenate(
        [jnp.transpose(r[:, 0:1] - qt[:, 0:1]) for r in rel], axis=0)
    ny = jnp.concatenate(
        [jnp.transpose(r[:, 1:2] - qt[:, 1:2]) for r in rel], axis=0)
    nz = jnp.concatenate(
        [jnp.transpose(r[:, 2:3] - qt[:, 2:3]) for r in rel], axis=0)

    den = jnp.abs(nx) + jnp.abs(ny)
    a = nx / jnp.where(den == 0.0, 1.0, den)
    p = jnp.where(ny >= 0.0, 1.0 - a, a - 1.0)
    p = jnp.where(den == 0.0, 0.0, p)

    ranks = []
    for i in range(_G):
        pi = p[i:i + 1]
        r_i = jnp.sum((p < pi).astype(jnp.int32), axis=0, keepdims=True)
        if i > 0:
            r_i = r_i + jnp.sum((p[0:i] == pi).astype(jnp.int32),
                                axis=0, keepdims=True)
        ranks.append(r_i)
    rank = jnp.concatenate(ranks, axis=0)

    def permute(v):
        rows = []
        for r in range(_G):
            selr = rank == r
            rows.append(jnp.sum(jnp.where(selr, v, 0.0), axis=0,
                                keepdims=True))
        return jnp.concatenate(rows, axis=0)

    sx, sy, sz = permute(nx), permute(ny), permute(nz)
    rx = jnp.concatenate([sx[1:], sx[0:1]], axis=0)
    ry = jnp.concatenate([sy[1:], sy[0:1]], axis=0)
    rz = jnp.concatenate([sz[1:], sz[0:1]], axis=0)

    cxn = sy * rz - sz * ry
    cyn = sz * rx - sx * rz
    czn = sx * ry - sy * rx
    nrm = jnp.sqrt(cxn * cxn + cyn * cyn + czn * czn)
    ux = cxn / nrm
    uy = cyn / nrm
    uz = czn / nrm
    posm = jnp.where(ux[0:1] > 0.0, 1.0, -1.0)
    gx, gy, gz = ux * posm, uy * posm, uz * posm

    ccx = (sx + rx) / 3.0
    ccy = (sy + ry) / 3.0
    ccz = (sz + rz) / 3.0

    rho = jnp.sqrt(ccx * ccx + ccy * ccy + ccz * ccz)
    rho_safe = jnp.where(rho == 0.0, 1.0, rho)
    ratio = jnp.clip(ccz / rho_safe, -1.0, 1.0)
    acos = jnp.arctan2(jnp.sqrt(jnp.maximum((1.0 - ratio) * (1.0 + ratio),
                                            0.0)), ratio)
    theta = jnp.where(rho == 0.0, 0.0, acos) * (1.0 / jnp.pi)
    phi = jnp.arctan2(ccy, ccx) * (1.0 / (2.0 * jnp.pi)) + 0.5

    nanm = (gx != gx) | (gy != gy) | (gz != gz)
    slot = jax.lax.broadcasted_iota(jnp.int32, (_G, _QB), 0)
    first = jnp.min(jnp.where(nanm, _G, slot), axis=0, keepdims=True)
    first = jnp.where(first == _G, 0, first)
    fsel = slot == first

    def first_val(v):
        return jnp.sum(jnp.where(fsel, v, 0.0), axis=0, keepdims=True)

    gx = jnp.where(nanm, first_val(gx), gx)
    gy = jnp.where(nanm, first_val(gy), gy)
    gz = jnp.where(nanm, first_val(gz), gz)
    ccx = jnp.where(nanm, first_val(ccx), ccx)
    ccy = jnp.where(nanm, first_val(ccy), ccy)
    ccz = jnp.where(nanm, first_val(ccz), ccz)

    chans = (ccx, ccy, ccz, rho, theta, phi, gx, gy, gz)
    for c, arr in enumerate(chans):
        out_ref[c, 0] = arr


def _mlp_kernel(feat_ref, w1_ref, g1_ref, b1_ref, w2_ref, cb2_ref, g2_ref,
                b2_ref, w3_ref, cb3_ref, out_ref):
    f = feat_ref[...]

    def conv(w, x):
        return jax.lax.dot_general(
            w.astype(jnp.bfloat16), x.astype(jnp.bfloat16),
            (((1,), (0,)), ((), ())),
            preferred_element_type=jnp.float32)

    def bn(x, g, b):
        m = jnp.mean(x, axis=1, keepdims=True)
        v = jnp.mean((x - m) * (x - m), axis=1, keepdims=True)
        xn = (x - m) / jnp.sqrt(v + 1e-5)
        return xn * g + b

    x = jax.nn.relu(bn(conv(w1_ref[...], f), g1_ref[...], b1_ref[...]))
    x = jax.nn.relu(bn(conv(w2_ref[...], x) + cb2_ref[...],
                       g2_ref[...], b2_ref[...]))
    x = conv(w3_ref[...], x) + cb3_ref[...]

    for b in range(_B):
        acc = x[:, b * _G * _N:(b * _G) * _N + _N]
        for g in range(1, _G):
            base = (b * _G + g) * _N
            acc = acc + x[:, base:base + _N]
        out_ref[b] = acc


@jax.jit
def kernel(center, conv1_w, bn1_g, bn1_b, conv2_w, conv2_b, bn2_g, bn2_b,
           conv3_w, conv3_b):
    feat = pl.pallas_call(
        _knn_feat_kernel,
        grid=(_B, _N // _QB),
        in_specs=[pl.BlockSpec((1, 3, _N), lambda b, q: (b, 0, 0)),
                  pl.BlockSpec((1, 3, _QB), lambda b, q: (b, 0, q))],
        out_specs=pl.BlockSpec((_C, 1, _G, _QB), lambda b, q: (0, b, 0, q)),
        out_shape=jax.ShapeDtypeStruct((_C, _B, _G, _N), jnp.float32),
    )(center, center)

    feat2 = feat.reshape(_C, _B * _G * _N)
    full = lambda s: pl.BlockSpec(s, lambda: tuple(0 for _ in s))
    out = pl.pallas_call(
        _mlp_kernel,
        grid=(),
        in_specs=[full((_C, _B * _G * _N))] + [full((_C, _C)), full((_C, 1)),
                  full((_C, 1)), full((_C, _C)), full((_C, 1)), full((_C, 1)),
                  full((_C, 1)), full((_C, _C)), full((_C, 1))],
        out_specs=full((_B, _C, _N)),
        out_shape=jax.ShapeDtypeStruct((_B, _C, _N), jnp.float32),
    )(feat2, conv1_w, bn1_g.reshape(_C, 1), bn1_b.reshape(_C, 1),
      conv2_w, conv2_b.reshape(_C, 1), bn2_g.reshape(_C, 1),
      bn2_b.reshape(_C, 1), conv3_w, conv3_b.reshape(_C, 1))
    return out


# speedup vs baseline: 2.3946x; 1.0969x over previous
import functools

import jax
import jax.numpy as jnp
from jax.experimental import pallas as pl

_K = 9
_G = 8
_QB = 256
_N = 4096
_B = 4
_C = 9


def _knn_feat_kernel(center_ref, qblk_ref, out_ref):
    keys = center_ref[0]
    sqk = jnp.sum(keys * keys, axis=0, keepdims=True)

    q = qblk_ref[0]
    sqq = jnp.sum(q * q, axis=0, keepdims=True)
    sqq_t = jnp.transpose(sqq)

    dqk = jax.lax.dot_general(
        q.astype(jnp.bfloat16), keys.astype(jnp.bfloat16),
        (((0,), (0,)), ((), ())),
        preferred_element_type=jnp.float32)
    dist = (sqq_t + sqk) - 2.0 * dqk

    iota = jax.lax.broadcasted_iota(jnp.int32, (_QB, _N), 1)
    big = jnp.float32(3e38)

    hi = keys.astype(jnp.bfloat16).astype(jnp.float32)
    r1 = keys - hi
    mid = r1.astype(jnp.bfloat16).astype(jnp.float32)
    r2 = r1 - mid
    k_t = jnp.transpose(
        jnp.concatenate([hi, mid, r2], axis=0)).astype(jnp.bfloat16)

    def extract(ohb):
        c = jax.lax.dot_general(ohb, k_t, (((1,), (0,)), ((), ())),
                                preferred_element_type=jnp.float32)
        return c[:, 0:3] + c[:, 3:6] + c[:, 6:9]

    fiota = iota.astype(jnp.float32)

    rel = []
    for t in range(_K):
        m = jnp.min(dist, axis=1, keepdims=True)
        cand = jnp.where(dist == m, fiota, jnp.float32(_N))
        jf = jnp.min(cand, axis=1, keepdims=True)
        ohf = jnp.where(cand == jf, 1.0, 0.0)
        if t > 0:
            rel.append(extract(ohf.astype(jnp.bfloat16)))
        if t < _K - 1:
            dist = dist + ohf * big

    qt = jnp.transpose(q)
    nx = jnp.concatenate(
        [jnp.transpose(r[:, 0:1] - qt[:, 0:1]) for r in rel], axis=0)
    ny = jnp.concatenate(
        [jnp.transpose(r[:, 1:2] - qt[:, 1:2]) for r in rel], axis=0)
    nz = jnp.concatenate(
        [jnp.transpose(r[:, 2:3] - qt[:, 2:3]) for r in rel], axis=0)

    den = jnp.abs(nx) + jnp.abs(ny)
    a = nx / jnp.where(den == 0.0, 1.0, den)
    p = jnp.where(ny >= 0.0, 1.0 - a, a - 1.0)
    p = jnp.where(den == 0.0, 0.0, p)

    ranks = []
    for i in range(_G):
        pi = p[i:i + 1]
        r_i = jnp.sum((p < pi).astype(jnp.int32), axis=0, keepdims=True)
        if i > 0:
            r_i = r_i + jnp.sum((p[0:i] == pi).astype(jnp.int32),
                                axis=0, keepdims=True)
        ranks.append(r_i)
    rank = jnp.concatenate(ranks, axis=0)

    def permute(v):
        rows = []
        for r in range(_G):
            selr = rank == r
            rows.append(jnp.sum(jnp.where(selr, v, 0.0), axis=0,
                                keepdims=True))
        return jnp.concatenate(rows, axis=0)

    sx, sy, sz = permute(nx), permute(ny), permute(nz)
    rx = jnp.concatenate([sx[1:], sx[0:1]], axis=0)
    ry = jnp.concatenate([sy[1:], sy[0:1]], axis=0)
    rz = jnp.concatenate([sz[1:], sz[0:1]], axis=0)

    cxn = sy * rz - sz * ry
    cyn = sz * rx - sx * rz
    czn = sx * ry - sy * rx
    nrm = jnp.sqrt(cxn * cxn + cyn * cyn + czn * czn)
    ux = cxn / nrm
    uy = cyn / nrm
    uz = czn / nrm
    posm = jnp.where(ux[0:1] > 0.0, 1.0, -1.0)
    gx, gy, gz = ux * posm, uy * posm, uz * posm

    ccx = (sx + rx) / 3.0
    ccy = (sy + ry) / 3.0
    ccz = (sz + rz) / 3.0

    rho = jnp.sqrt(ccx * ccx + ccy * ccy + ccz * ccz)
    rho_safe = jnp.where(rho == 0.0, 1.0, rho)
    ratio = jnp.clip(ccz / rho_safe, -1.0, 1.0)
    acos = jnp.arctan2(jnp.sqrt(jnp.maximum((1.0 - ratio) * (1.0 + ratio),
                                            0.0)), ratio)
    theta = jnp.where(rho == 0.0, 0.0, acos) * (1.0 / jnp.pi)
    phi = jnp.arctan2(ccy, ccx) * (1.0 / (2.0 * jnp.pi)) + 0.5

    nanm = (gx != gx) | (gy != gy) | (gz != gz)
    slot = jax.lax.broadcasted_iota(jnp.int32, (_G, _QB), 0)
    first = jnp.min(jnp.where(nanm, _G, slot), axis=0, keepdims=True)
    first = jnp.where(first == _G, 0, first)
    fsel = slot == first

    def first_val(v):
        return jnp.sum(jnp.where(fsel, v, 0.0), axis=0, keepdims=True)

    gx = jnp.where(nanm, first_val(gx), gx)
    gy = jnp.where(nanm, first_val(gy), gy)
    gz = jnp.where(nanm, first_val(gz), gz)
    ccx = jnp.where(nanm, first_val(ccx), ccx)
    ccy = jnp.where(nanm, first_val(ccy), ccy)
    ccz = jnp.where(nanm, first_val(ccz), ccz)

    chans = (ccx, ccy, ccz, rho, theta, phi, gx, gy, gz)
    for c, arr in enumerate(chans):
        out_ref[c, 0] = arr


def _mlp_kernel(feat_ref, w1_ref, g1_ref, b1_ref, w2_ref, cb2_ref, g2_ref,
                b2_ref, w3_ref, cb3_ref, out_ref):
    f = feat_ref[...]

    def conv(w, x):
        return jax.lax.dot_general(
            w.astype(jnp.bfloat16), x.astype(jnp.bfloat16),
            (((1,), (0,)), ((), ())),
            preferred_element_type=jnp.float32)

    def bn(x, g, b):
        m = jnp.mean(x, axis=1, keepdims=True)
        v = jnp.mean((x - m) * (x - m), axis=1, keepdims=True)
        xn = (x - m) / jnp.sqrt(v + 1e-5)
        return xn * g + b

    x = jax.nn.relu(bn(conv(w1_ref[...], f), g1_ref[...], b1_ref[...]))
    x = jax.nn.relu(bn(conv(w2_ref[...], x) + cb2_ref[...],
                       g2_ref[...], b2_ref[...]))
    x = conv(w3_ref[...], x) + cb3_ref[...]

    for b in range(_B):
        acc = x[:, b * _G * _N:(b * _G) * _N + _N]
        for g in range(1, _G):
            base = (b * _G + g) * _N
            acc = acc + x[:, base:base + _N]
        out_ref[b] = acc


@jax.jit
def kernel(center, conv1_w, bn1_g, bn1_b, conv2_w, conv2_b, bn2_g, bn2_b,
           conv3_w, conv3_b):
    feat = pl.pallas_call(
        _knn_feat_kernel,
        grid=(_B, _N // _QB),
        in_specs=[pl.BlockSpec((1, 3, _N), lambda b, q: (b, 0, 0)),
                  pl.BlockSpec((1, 3, _QB), lambda b, q: (b, 0, q))],
        out_specs=pl.BlockSpec((_C, 1, _G, _QB), lambda b, q: (0, b, 0, q)),
        out_shape=jax.ShapeDtypeStruct((_C, _B, _G, _N), jnp.float32),
    )(center, center)

    feat2 = feat.reshape(_C, _B * _G * _N)
    full = lambda s: pl.BlockSpec(s, lambda: tuple(0 for _ in s))
    out = pl.pallas_call(
        _mlp_kernel,
        grid=(),
        in_specs=[full((_C, _B * _G * _N))] + [full((_C, _C)), full((_C, 1)),
                  full((_C, 1)), full((_C, _C)), full((_C, 1)), full((_C, 1)),
                  full((_C, 1)), full((_C, _C)), full((_C, 1))],
        out_specs=full((_B, _C, _N)),
        out_shape=jax.ShapeDtypeStruct((_B, _C, _N), jnp.float32),
    )(feat2, conv1_w, bn1_g.reshape(_C, 1), bn1_b.reshape(_C, 1),
      conv2_w, conv2_b.reshape(_C, 1), bn2_g.reshape(_C, 1),
      bn2_b.reshape(_C, 1), conv3_w, conv3_b.reshape(_C, 1))
    return out
